# baseline (speedup 1.0000x reference)
import functools

import jax
import jax.numpy as jnp
from jax import lax
from jax.experimental import pallas as pl
from jax.experimental.pallas import tpu as pltpu
from jax.experimental.pallas import tpu_sc as plsc

NUM_EMBEDDINGS = 1000000
EMBEDDING_DIM = 64
BATCH = 16384
SEQ = 50

_NC = 2
_NS = 16
_NW = _NC * _NS

_ROWS_PER_W = BATCH // _NW
_IDS_PER_W = _ROWS_PER_W * SEQ
_G = 8
_CHUNK = _G * SEQ
_N_CHUNKS = _ROWS_PER_W // _G
_NBUF = 4
_N_ROUNDS = _N_CHUNKS // _NBUF

assert _ROWS_PER_W * _NW == BATCH
assert _G * _N_CHUNKS == _ROWS_PER_W
assert _NBUF * _N_ROUNDS == _N_CHUNKS


def _body(ids_hbm, table_hbm, out_hbm, idx_v, rows_v, *sems):
    gsem = sems[:_NBUF]
    osem = sems[_NBUF:]
    wid = lax.axis_index("s") * _NC + lax.axis_index("c")
    base = wid * _ROWS_PER_W

    pltpu.sync_copy(ids_hbm.at[pl.ds(wid * _IDS_PER_W, _IDS_PER_W)], idx_v)

    for b in range(_NBUF):
        pltpu.async_copy(
            table_hbm.at[idx_v.at[pl.ds(b * _CHUNK, _CHUNK)]],
            rows_v.at[b], gsem[b])

    def round_body(r, carry):
        for b in range(_NBUF):
            c = r * _NBUF + b
            pltpu.make_async_copy(
                table_hbm.at[idx_v.at[pl.ds(c * _CHUNK, _CHUNK)]],
                rows_v.at[b], gsem[b]).wait()
            for j in range(_G):
                pltpu.async_copy(
                    rows_v.at[b, pl.ds(j * SEQ, SEQ)],
                    out_hbm.at[base + c * _G + j], osem[b])
            for j in range(_G):
                pltpu.make_async_copy(
                    rows_v.at[b, pl.ds(j * SEQ, SEQ)],
                    out_hbm.at[base + c * _G + j], osem[b]).wait()
            nxt = c + _NBUF

            @pl.when(nxt < _N_CHUNKS)
            def _():
                pltpu.async_copy(
                    table_hbm.at[idx_v.at[pl.ds(nxt * _CHUNK, _CHUNK)]],
                    rows_v.at[b], gsem[b])

        return carry

    lax.fori_loop(0, _N_ROUNDS, round_body, 0)


@jax.jit
def kernel(token_ids, weight):
    ids = token_ids.reshape(BATCH * SEQ).astype(jnp.int32)
    run = pl.kernel(
        _body,
        out_type=jax.ShapeDtypeStruct((BATCH, SEQ, EMBEDDING_DIM), jnp.float32),
        mesh=plsc.VectorSubcoreMesh(
            core_axis_name="c", subcore_axis_name="s",
            num_cores=_NC, num_subcores=_NS),
        compiler_params=pltpu.CompilerParams(use_tc_tiling_on_sc=False),
        scratch_types=[
            pltpu.VMEM((_IDS_PER_W,), jnp.int32),
            pltpu.VMEM((_NBUF, _CHUNK, EMBEDDING_DIM), jnp.float32),
        ] + [pltpu.SemaphoreType.DMA] * (2 * _NBUF),
    )
    return run(ids, weight)
